# Initial kernel scaffold; baseline (speedup 1.0000x reference)
import jax
import jax.numpy as jnp
from jax.experimental import pallas as pl


def kernel(x, edge_index, fp_W, fp_b, fp_g, fp_beta, ip_W, ip_b, gat_W, gat_asrc, gat_adst, gat_bias, proj_W, proj_b, ln_g, ln_b):
    raise NotImplementedError("write your pallas kernel here")


# speedup vs baseline: 13.3948x; 13.3948x over previous
import functools

import jax
import jax.numpy as jnp
from jax.experimental import pallas as pl
from jax.experimental.pallas import tpu as pltpu

N = 50000
E = 800000
D_IN = 3
HID = 64
HEADS = 4
HD = 16
LAYERS = 3

BLK = 2000
GRID = N // BLK

_INTERP = False


def _ln(h, g, b):
    m = jnp.mean(h, axis=-1, keepdims=True)
    v = jnp.mean((h - m) ** 2, axis=-1, keepdims=True)
    return (h - m) * jax.lax.rsqrt(v + 1e-5) * g + b


def _head_logits(hp, acat):
    t = hp * acat
    return jnp.concatenate(
        [jnp.sum(t[:, k * HD:(k + 1) * HD], axis=1, keepdims=True) for k in range(HEADS)],
        axis=1)


def _dense_in_body(x_ref, fpW, fpb, fpg, fpbeta, ipW, ipb, W0, as0, ad0,
                   h_out, hA, hB, es4, ed4):
    x = x_ref[...]
    h = jnp.dot(x, fpW[...], preferred_element_type=jnp.float32) + fpb[...]
    h = _ln(h, fpg[...], fpbeta[...])
    h = jnp.where(h > 0, h, jnp.exp(jnp.minimum(h, 0.0)) - 1.0)
    h = jnp.dot(h, ipW[...], preferred_element_type=jnp.float32) + ipb[...]
    h_out[...] = h
    hp = jnp.dot(h, W0[...], preferred_element_type=jnp.float32)
    hA[...] = hp[:, :32]
    hB[...] = hp[:, 32:]
    es4[...] = _head_logits(hp, as0[...])
    ed4[...] = _head_logits(hp, ad0[...])


def _dense_layer_body(h_ref, accA, accB, d0, d1, d2, d3, bias, projW, projb,
                      lng, lnb, Wn, asn, adn,
                      h_out, hA=None, hB=None, es4=None, ed4=None, *, last):
    acc = jnp.concatenate([accA[...], accB[...]], axis=1)
    dref = (d0, d1, d2, d3)
    parts = []
    for k in range(HEADS):
        dk = dref[k][...]
        parts.append(acc[:, k * HD:(k + 1) * HD] * (1.0 / (dk + 1e-16)))
    mh = jnp.concatenate(parts, axis=1) + bias[...]
    out = jnp.dot(mh, projW[...], preferred_element_type=jnp.float32) + projb[...]
    h = _ln(out + h_ref[...], lng[...], lnb[...])
    h_out[...] = h
    if not last:
        hp = jnp.dot(h, Wn[...], preferred_element_type=jnp.float32)
        hA[...] = hp[:, :32]
        hB[...] = hp[:, 32:]
        es4[...] = _head_logits(hp, asn[...])
        ed4[...] = _head_logits(hp, adn[...])


def _full(shape):
    return pl.BlockSpec(shape, lambda i: tuple(0 for _ in shape))


def _rows(width):
    return pl.BlockSpec((BLK, width), lambda i: (i, 0))


def _dense_in(x, fpW, fpb, fpg, fpbeta, ipW, ipb, W0, as0, ad0):
    out_shapes = [
        jax.ShapeDtypeStruct((N, HID), jnp.float32),
        jax.ShapeDtypeStruct((N, 32), jnp.float32),
        jax.ShapeDtypeStruct((N, 32), jnp.float32),
        jax.ShapeDtypeStruct((N, HEADS), jnp.float32),
        jax.ShapeDtypeStruct((N, HEADS), jnp.float32),
    ]
    return pl.pallas_call(
        _dense_in_body,
        grid=(GRID,),
        in_specs=[_rows(D_IN), _full((D_IN, HID)), _full((HID,)), _full((HID,)),
                  _full((HID,)), _full((HID, HID)), _full((HID,)),
                  _full((HID, HID)), _full((HID,)), _full((HID,))],
        out_specs=[_rows(HID), _rows(32), _rows(32), _rows(HEADS), _rows(HEADS)],
        out_shape=out_shapes,
        interpret=_INTERP,
    )(x, fpW, fpb, fpg, fpbeta, ipW, ipb, W0, as0, ad0)


def _dense_layer(h, accA, accB, d4, bias, projW, projb, lng, lnb, Wn, asn, adn, last):
    out_shapes = [jax.ShapeDtypeStruct((N, HID), jnp.float32)]
    out_specs = [_rows(HID)]
    if not last:
        out_shapes += [
            jax.ShapeDtypeStruct((N, 32), jnp.float32),
            jax.ShapeDtypeStruct((N, 32), jnp.float32),
            jax.ShapeDtypeStruct((N, HEADS), jnp.float32),
            jax.ShapeDtypeStruct((N, HEADS), jnp.float32),
        ]
        out_specs += [_rows(32), _rows(32), _rows(HEADS), _rows(HEADS)]
    d0, d1, d2, d3 = (d4[k].reshape(N, 1) for k in range(HEADS))
    return pl.pallas_call(
        functools.partial(_dense_layer_body, last=last),
        grid=(GRID,),
        in_specs=[_rows(HID), _rows(32), _rows(32),
                  _rows(1), _rows(1), _rows(1), _rows(1),
                  _full((HID,)), _full((HID, HID)), _full((HID,)),
                  _full((HID,)), _full((HID,)),
                  _full((HID, HID)), _full((HID,)), _full((HID,))],
        out_specs=out_specs,
        out_shape=out_shapes,
        interpret=_INTERP,
    )(h, accA, accB, d0, d1, d2, d3, bias, projW, projb, lng, lnb, Wn, asn, adn)


def _edges(hA, hB, es4, ed4, src, dst):
    e = es4[src] + ed4[dst]
    e = jnp.where(e > 0, e, 0.2 * e)
    w = jnp.exp(e)
    den = jax.ops.segment_sum(w, dst, num_segments=N)
    hp = jnp.concatenate([hA, hB], axis=1)
    wr = jnp.repeat(w, HD, axis=1)
    acc = jax.ops.segment_sum(wr * hp[src], dst, num_segments=N)
    return acc[:, :32], acc[:, 32:], [den[:, k] for k in range(HEADS)]


def kernel(x, edge_index, fp_W, fp_b, fp_g, fp_beta, ip_W, ip_b,
           gat_W, gat_asrc, gat_adst, gat_bias, proj_W, proj_b, ln_g, ln_b):
    src = edge_index[0]
    dst = edge_index[1]
    Wc = [gat_W[l].transpose(1, 0, 2).reshape(HID, HEADS * HD) for l in range(LAYERS)]
    asc = [gat_asrc[l].reshape(HEADS * HD) for l in range(LAYERS)]
    adc = [gat_adst[l].reshape(HEADS * HD) for l in range(LAYERS)]
    bc = [gat_bias[l].reshape(HEADS * HD) for l in range(LAYERS)]

    h, hA, hB, es4, ed4 = _dense_in(x, fp_W, fp_b, fp_g, fp_beta, ip_W, ip_b,
                                    Wc[0], asc[0], adc[0])
    for l in range(LAYERS):
        accA, accB, d4 = _edges(hA, hB, es4, ed4, src, dst)
        last = l == LAYERS - 1
        nxt = l + 1 if not last else l
        res = _dense_layer(h, accA, accB, d4, bc[l], proj_W[l], proj_b[l],
                           ln_g[l], ln_b[l], Wc[nxt], asc[nxt], adc[nxt], last)
        if last:
            (h,) = res
        else:
            h, hA, hB, es4, ed4 = res
    return h


# speedup vs baseline: 58.7118x; 4.3832x over previous
import functools

import jax
import jax.numpy as jnp
from jax import lax
from jax.experimental import pallas as pl
from jax.experimental.pallas import tpu as pltpu
from jax.experimental.pallas import tpu_sc as plsc

N = 50000
E = 800000
D_IN = 3
HID = 64
HEADS = 4
HD = 16
LAYERS = 3

BLK = 2000
GRID = N // BLK

_INTERP = False


def _ln(h, g, b):
    m = jnp.mean(h, axis=-1, keepdims=True)
    v = jnp.mean((h - m) ** 2, axis=-1, keepdims=True)
    return (h - m) * jax.lax.rsqrt(v + 1e-5) * g + b


def _head_logits(hp, acat):
    t = hp * acat
    return jnp.concatenate(
        [jnp.sum(t[:, k * HD:(k + 1) * HD], axis=1, keepdims=True) for k in range(HEADS)],
        axis=1)


def _dense_in_body(x_ref, fpW, fpb, fpg, fpbeta, ipW, ipb, W0, as0, ad0,
                   h_out, hA, hB, es4, ed4):
    x = x_ref[...]
    h = jnp.dot(x, fpW[...], preferred_element_type=jnp.float32) + fpb[...]
    h = _ln(h, fpg[...], fpbeta[...])
    h = jnp.where(h > 0, h, jnp.exp(jnp.minimum(h, 0.0)) - 1.0)
    h = jnp.dot(h, ipW[...], preferred_element_type=jnp.float32) + ipb[...]
    h_out[...] = h
    hp = jnp.dot(h, W0[...], preferred_element_type=jnp.float32)
    hA[...] = hp[:, :32]
    hB[...] = hp[:, 32:]
    es4[...] = _head_logits(hp, as0[...])
    ed4[...] = _head_logits(hp, ad0[...])


def _dense_layer_body(h_ref, accA, accB, d0, d1, d2, d3, bias, projW, projb,
                      lng, lnb, Wn, asn, adn,
                      h_out, hA=None, hB=None, es4=None, ed4=None, *, last):
    acc = jnp.concatenate([accA[...], accB[...]], axis=1)
    dref = (d0, d1, d2, d3)
    parts = []
    for k in range(HEADS):
        dk = dref[k][...]
        parts.append(acc[:, k * HD:(k + 1) * HD] * (1.0 / (dk + 1e-16)))
    mh = jnp.concatenate(parts, axis=1) + bias[...]
    out = jnp.dot(mh, projW[...], preferred_element_type=jnp.float32) + projb[...]
    h = _ln(out + h_ref[...], lng[...], lnb[...])
    h_out[...] = h
    if not last:
        hp = jnp.dot(h, Wn[...], preferred_element_type=jnp.float32)
        hA[...] = hp[:, :32]
        hB[...] = hp[:, 32:]
        es4[...] = _head_logits(hp, asn[...])
        ed4[...] = _head_logits(hp, adn[...])


def _full(shape):
    return pl.BlockSpec(shape, lambda i: tuple(0 for _ in shape))


def _rows(width):
    return pl.BlockSpec((BLK, width), lambda i: (i, 0))


def _dense_in(x, fpW, fpb, fpg, fpbeta, ipW, ipb, W0, as0, ad0):
    out_shapes = [
        jax.ShapeDtypeStruct((N, HID), jnp.float32),
        jax.ShapeDtypeStruct((N, 32), jnp.float32),
        jax.ShapeDtypeStruct((N, 32), jnp.float32),
        jax.ShapeDtypeStruct((N, HEADS), jnp.float32),
        jax.ShapeDtypeStruct((N, HEADS), jnp.float32),
    ]
    return pl.pallas_call(
        _dense_in_body,
        grid=(GRID,),
        in_specs=[_rows(D_IN), _full((D_IN, HID)), _full((HID,)), _full((HID,)),
                  _full((HID,)), _full((HID, HID)), _full((HID,)),
                  _full((HID, HID)), _full((HID,)), _full((HID,))],
        out_specs=[_rows(HID), _rows(32), _rows(32), _rows(HEADS), _rows(HEADS)],
        out_shape=out_shapes,
        interpret=_INTERP,
    )(x, fpW, fpb, fpg, fpbeta, ipW, ipb, W0, as0, ad0)


def _dense_layer(h, accA, accB, d4, bias, projW, projb, lng, lnb, Wn, asn, adn, last):
    out_shapes = [jax.ShapeDtypeStruct((N, HID), jnp.float32)]
    out_specs = [_rows(HID)]
    if not last:
        out_shapes += [
            jax.ShapeDtypeStruct((N, 32), jnp.float32),
            jax.ShapeDtypeStruct((N, 32), jnp.float32),
            jax.ShapeDtypeStruct((N, HEADS), jnp.float32),
            jax.ShapeDtypeStruct((N, HEADS), jnp.float32),
        ]
        out_specs += [_rows(32), _rows(32), _rows(HEADS), _rows(HEADS)]
    d0, d1, d2, d3 = (d4[k].reshape(N, 1) for k in range(HEADS))
    return pl.pallas_call(
        functools.partial(_dense_layer_body, last=last),
        grid=(GRID,),
        in_specs=[_rows(HID), _rows(32), _rows(32),
                  _rows(1), _rows(1), _rows(1), _rows(1),
                  _full((HID,)), _full((HID, HID)), _full((HID,)),
                  _full((HID,)), _full((HID,)),
                  _full((HID, HID)), _full((HID,)), _full((HID,))],
        out_specs=out_specs,
        out_shape=out_shapes,
        interpret=_INTERP,
    )(h, accA, accB, d0, d1, d2, d3, bias, projW, projb, lng, lnb, Wn, asn, adn)


def _build_edge_kernel(n, e_real, rows_pad, chunks, npad_acc, npad_den, zcopies,
                       interpret=False):
    ns = 16
    rows_chunk = 2
    ce = rows_chunk * 128
    nrt = npad_acc // ns
    zrows = nrt // zcopies
    dent = npad_den // ns
    zd_sz = ((dent + 15) // 16) * 16

    def body(srcr, dstr, es0, es1, es2, es3, ed0, ed1, ed2, ed3, hA, hB,
             accA, accB, d0, d1, d2, d3,
             idx_s, idx_d, esa, esb, eda, edb, h_buf, w0, w1, out_buf,
             acc_sh, den0_sh, den1_sh, sem):
        c = lax.axis_index("c")
        s = lax.axis_index("s")
        zero16 = lax.broadcast(jnp.float32(0), (16,))

        def zr_body(r, _):
            h_buf[r, pl.ds(0, 16)] = zero16
            h_buf[r, pl.ds(16, 16)] = zero16
            return 0
        lax.fori_loop(0, ce, zr_body, 0)

        def zd_body(j, _):
            w0[pl.ds(j * 16, 16)] = zero16
            return 0
        lax.fori_loop(0, ce // 16, zd_body, 0)

        for t in range(zcopies):
            pltpu.sync_copy(h_buf.at[pl.ds(0, zrows), :],
                            acc_sh.at[pl.ds(s * nrt + t * zrows, zrows), :])
        for t in range(dent // 128):
            pltpu.sync_copy(w0.at[pl.ds(0, 128)],
                            den0_sh.at[pl.ds(s * dent + t * 128, 128)])
            pltpu.sync_copy(w0.at[pl.ds(0, 128)],
                            den1_sh.at[pl.ds(s * dent + t * 128, 128)])
        plsc.subcore_barrier()

        def chunk_body(m, _):
            base = (m * ns + s) * rows_chunk
            pltpu.sync_copy(srcr.at[pl.ds(base, rows_chunk), :], idx_s)
            pltpu.sync_copy(dstr.at[pl.ds(base, rows_chunk), :], idx_d)

            def gather_phase(tes_a, tes_b, ted_a, ted_b, t_h):
                cps = []
                for j in range(rows_chunk):
                    sl = pl.ds(j * 128, 128)
                    cps.append(pltpu.async_copy(tes_a.at[idx_s.at[j]], esa.at[sl], sem))
                    cps.append(pltpu.async_copy(tes_b.at[idx_s.at[j]], esb.at[sl], sem))
                    cps.append(pltpu.async_copy(ted_a.at[idx_d.at[j]], eda.at[sl], sem))
                    cps.append(pltpu.async_copy(ted_b.at[idx_d.at[j]], edb.at[sl], sem))
                    cps.append(pltpu.async_copy(t_h.at[idx_s.at[j]], h_buf.at[sl, :], sem))
                for cp in cps:
                    cp.wait()

            @pl.when(c == 0)
            def _():
                gather_phase(es0, es1, ed0, ed1, hA)

            @pl.when(c == 1)
            def _():
                gather_phase(es2, es3, ed2, ed3, hB)

            base_e = base * 128
            for ebuf_s, ebuf_d, wbuf in ((esa, eda, w0), (esb, edb, w1)):
                def w_body(j, _, ebuf_s=ebuf_s, ebuf_d=ebuf_d, wbuf=wbuf):
                    sl = pl.ds(j * 16, 16)
                    e = ebuf_s[sl] + ebuf_d[sl]
                    e = jnp.where(e > 0, e, 0.2 * e)
                    w = jnp.exp(e)
                    ge = base_e + j * 16 + lax.iota(jnp.int32, 16)
                    wbuf[sl] = jnp.where(ge < e_real, w, 0.0)
                    return 0
                lax.fori_loop(0, ce // 16, w_body, 0)

            def s_body(g, _):
                wv0 = w0[pl.ds(g * 16, 16)]
                wv1 = w1[pl.ds(g * 16, 16)]
                for i in range(16):
                    e = g * 16 + i
                    w0s = lax.broadcast(wv0[i], (16,))
                    w1s = lax.broadcast(wv1[i], (16,))
                    out_buf[e, pl.ds(0, 16)] = w0s * h_buf[e, pl.ds(0, 16)]
                    out_buf[e, pl.ds(16, 16)] = w1s * h_buf[e, pl.ds(16, 16)]
                return 0
            lax.fori_loop(0, ce // 16, s_body, 0)

            cps = []
            for j in range(rows_chunk):
                sl = pl.ds(j * 128, 128)
                cps.append(pltpu.async_copy(out_buf.at[sl, :], acc_sh.at[idx_d.at[j]], sem, add=True))
                cps.append(pltpu.async_copy(w0.at[sl], den0_sh.at[idx_d.at[j]], sem, add=True))
                cps.append(pltpu.async_copy(w1.at[sl], den1_sh.at[idx_d.at[j]], sem, add=True))
            for cp in cps:
                cp.wait()
            return 0

        lax.fori_loop(0, chunks, chunk_body, 0)
        plsc.subcore_barrier()

        @pl.when(c == 0)
        def _():
            pltpu.sync_copy(acc_sh.at[pl.ds(s * nrt, nrt), :], accA.at[pl.ds(s * nrt, nrt), :])
            pltpu.sync_copy(den0_sh.at[pl.ds(s * dent, dent)], d0.at[pl.ds(s * dent, dent)])
            pltpu.sync_copy(den1_sh.at[pl.ds(s * dent, dent)], d1.at[pl.ds(s * dent, dent)])

        @pl.when(c == 1)
        def _():
            pltpu.sync_copy(acc_sh.at[pl.ds(s * nrt, nrt), :], accB.at[pl.ds(s * nrt, nrt), :])
            pltpu.sync_copy(den0_sh.at[pl.ds(s * dent, dent)], d2.at[pl.ds(s * dent, dent)])
            pltpu.sync_copy(den1_sh.at[pl.ds(s * dent, dent)], d3.at[pl.ds(s * dent, dent)])

    f32 = jnp.float32
    out_type = [jax.ShapeDtypeStruct((npad_acc, 32), f32),
                jax.ShapeDtypeStruct((npad_acc, 32), f32)] + \
               [jax.ShapeDtypeStruct((npad_den,), f32)] * 4
    scratch_types = [
        pltpu.VMEM((rows_chunk, 128), jnp.int32),
        pltpu.VMEM((rows_chunk, 128), jnp.int32),
        pltpu.VMEM((ce,), f32),
        pltpu.VMEM((ce,), f32),
        pltpu.VMEM((ce,), f32),
        pltpu.VMEM((ce,), f32),
        pltpu.VMEM((ce, 32), f32),
        pltpu.VMEM((ce,), f32),
        pltpu.VMEM((ce,), f32),
        pltpu.VMEM((ce, 32), f32),
        pltpu.VMEM_SHARED((npad_acc, 32), f32),
        pltpu.VMEM_SHARED((npad_den,), f32),
        pltpu.VMEM_SHARED((npad_den,), f32),
        pltpu.SemaphoreType.DMA,
    ]
    mesh = plsc.VectorSubcoreMesh(core_axis_name="c", subcore_axis_name="s",
                                  num_cores=2, num_subcores=ns)
    return pl.kernel(body, out_type=out_type, mesh=mesh,
                     scratch_types=scratch_types,
                     compiler_params=pltpu.CompilerParams(use_tc_tiling_on_sc=False),
                     interpret=interpret)


_EDGE_ROWS_PAD = 6272
_CHUNKS = 196
_NPAD_ACC = 50048
_NPAD_DEN = 51200


def _edges(hA, hB, es4, ed4, srcr, dstr):
    fn = _build_edge_kernel(N, E, _EDGE_ROWS_PAD, _CHUNKS, _NPAD_ACC,
                            _NPAD_DEN, 17, interpret=_INTERP)
    es = [es4[:, k] for k in range(HEADS)]
    ed = [ed4[:, k] for k in range(HEADS)]
    accA, accB, d0, d1, d2, d3 = fn(srcr, dstr, es[0], es[1], es[2], es[3],
                                    ed[0], ed[1], ed[2], ed[3], hA, hB)
    return accA[:N], accB[:N], [d[:N] for d in (d0, d1, d2, d3)]


def kernel(x, edge_index, fp_W, fp_b, fp_g, fp_beta, ip_W, ip_b,
           gat_W, gat_asrc, gat_adst, gat_bias, proj_W, proj_b, ln_g, ln_b):
    pad = _EDGE_ROWS_PAD * 128 - E
    zpad = jnp.zeros((pad,), jnp.int32)
    srcr = jnp.concatenate([edge_index[0], zpad]).reshape(_EDGE_ROWS_PAD, 128)
    dstr = jnp.concatenate([edge_index[1], zpad]).reshape(_EDGE_ROWS_PAD, 128)
    Wc = [gat_W[l].transpose(1, 0, 2).reshape(HID, HEADS * HD) for l in range(LAYERS)]
    asc = [gat_asrc[l].reshape(HEADS * HD) for l in range(LAYERS)]
    adc = [gat_adst[l].reshape(HEADS * HD) for l in range(LAYERS)]
    bc = [gat_bias[l].reshape(HEADS * HD) for l in range(LAYERS)]

    h, hA, hB, es4, ed4 = _dense_in(x, fp_W, fp_b, fp_g, fp_beta, ip_W, ip_b,
                                    Wc[0], asc[0], adc[0])
    for l in range(LAYERS):
        accA, accB, d4 = _edges(hA, hB, es4, ed4, srcr, dstr)
        last = l == LAYERS - 1
        nxt = l + 1 if not last else l
        res = _dense_layer(h, accA, accB, d4, bc[l], proj_W[l], proj_b[l],
                           ln_g[l], ln_b[l], Wc[nxt], asc[nxt], adc[nxt], last)
        if last:
            (h,) = res
        else:
            h, hA, hB, es4, ed4 = res
    return h


# speedup vs baseline: 101.4449x; 1.7278x over previous
import functools

import jax
import jax.numpy as jnp
from jax import lax
from jax.experimental import pallas as pl
from jax.experimental.pallas import tpu as pltpu
from jax.experimental.pallas import tpu_sc as plsc

N = 50000
E = 800000
D_IN = 3
HID = 64
HEADS = 4
HD = 16
LAYERS = 3

BLK = 2000
GRID = N // BLK

_INTERP = False


def _ln(h, g, b):
    m = jnp.mean(h, axis=-1, keepdims=True)
    v = jnp.mean((h - m) ** 2, axis=-1, keepdims=True)
    return (h - m) * jax.lax.rsqrt(v + 1e-5) * g + b


def _head_logits(hp, acat):
    t = hp * acat
    return jnp.concatenate(
        [jnp.sum(t[:, k * HD:(k + 1) * HD], axis=1, keepdims=True) for k in range(HEADS)],
        axis=1)


def _dense_in_body(x_ref, fpW, fpb, fpg, fpbeta, ipW, ipb, W0, as0, ad0,
                   h_out, hA, hB, es4, ed4):
    x = x_ref[...]
    h = jnp.dot(x, fpW[...], preferred_element_type=jnp.float32) + fpb[...]
    h = _ln(h, fpg[...], fpbeta[...])
    h = jnp.where(h > 0, h, jnp.exp(jnp.minimum(h, 0.0)) - 1.0)
    h = jnp.dot(h, ipW[...], preferred_element_type=jnp.float32) + ipb[...]
    h_out[...] = h
    hp = jnp.dot(h, W0[...], preferred_element_type=jnp.float32)
    hA[...] = hp[:, :32]
    hB[...] = hp[:, 32:]
    es4[...] = _head_logits(hp, as0[...])
    ed4[...] = _head_logits(hp, ad0[...])


def _dense_layer_body(h_ref, accA, accB, d0, d1, d2, d3, bias, projW, projb,
                      lng, lnb, Wn, asn, adn,
                      h_out, hA=None, hB=None, es4=None, ed4=None, *, last):
    acc = jnp.concatenate([accA[...], accB[...]], axis=1)
    dref = (d0, d1, d2, d3)
    parts = []
    for k in range(HEADS):
        dk = dref[k][...]
        parts.append(acc[:, k * HD:(k + 1) * HD] * (1.0 / (dk + 1e-16)))
    mh = jnp.concatenate(parts, axis=1) + bias[...]
    out = jnp.dot(mh, projW[...], preferred_element_type=jnp.float32) + projb[...]
    h = _ln(out + h_ref[...], lng[...], lnb[...])
    h_out[...] = h
    if not last:
        hp = jnp.dot(h, Wn[...], preferred_element_type=jnp.float32)
        hA[...] = hp[:, :32]
        hB[...] = hp[:, 32:]
        es4[...] = _head_logits(hp, asn[...])
        ed4[...] = _head_logits(hp, adn[...])


def _full(shape):
    return pl.BlockSpec(shape, lambda i: tuple(0 for _ in shape))


def _rows(width):
    return pl.BlockSpec((BLK, width), lambda i: (i, 0))


def _dense_in(x, fpW, fpb, fpg, fpbeta, ipW, ipb, W0, as0, ad0):
    out_shapes = [
        jax.ShapeDtypeStruct((N, HID), jnp.float32),
        jax.ShapeDtypeStruct((N, 32), jnp.float32),
        jax.ShapeDtypeStruct((N, 32), jnp.float32),
        jax.ShapeDtypeStruct((N, HEADS), jnp.float32),
        jax.ShapeDtypeStruct((N, HEADS), jnp.float32),
    ]
    return pl.pallas_call(
        _dense_in_body,
        grid=(GRID,),
        in_specs=[_rows(D_IN), _full((D_IN, HID)), _full((HID,)), _full((HID,)),
                  _full((HID,)), _full((HID, HID)), _full((HID,)),
                  _full((HID, HID)), _full((HID,)), _full((HID,))],
        out_specs=[_rows(HID), _rows(32), _rows(32), _rows(HEADS), _rows(HEADS)],
        out_shape=out_shapes,
        interpret=_INTERP,
    )(x, fpW, fpb, fpg, fpbeta, ipW, ipb, W0, as0, ad0)


def _dense_layer(h, accA, accB, d4, bias, projW, projb, lng, lnb, Wn, asn, adn, last):
    out_shapes = [jax.ShapeDtypeStruct((N, HID), jnp.float32)]
    out_specs = [_rows(HID)]
    if not last:
        out_shapes += [
            jax.ShapeDtypeStruct((N, 32), jnp.float32),
            jax.ShapeDtypeStruct((N, 32), jnp.float32),
            jax.ShapeDtypeStruct((N, HEADS), jnp.float32),
            jax.ShapeDtypeStruct((N, HEADS), jnp.float32),
        ]
        out_specs += [_rows(32), _rows(32), _rows(HEADS), _rows(HEADS)]
    d0, d1, d2, d3 = (d4[k].reshape(N, 1) for k in range(HEADS))
    return pl.pallas_call(
        functools.partial(_dense_layer_body, last=last),
        grid=(GRID,),
        in_specs=[_rows(HID), _rows(32), _rows(32),
                  _rows(1), _rows(1), _rows(1), _rows(1),
                  _full((HID,)), _full((HID, HID)), _full((HID,)),
                  _full((HID,)), _full((HID,)),
                  _full((HID, HID)), _full((HID,)), _full((HID,))],
        out_specs=out_specs,
        out_shape=out_shapes,
        interpret=_INTERP,
    )(h, accA, accB, d0, d1, d2, d3, bias, projW, projb, lng, lnb, Wn, asn, adn)


def _build_edge_kernel(n, e_real, rows_pad, chunks, npad_acc, npad_den):
    ns = 16
    ce = 128
    nbuf = 4
    nrt = npad_acc // ns
    dent = npad_den // ns
    assert chunks % nbuf == 0 and nrt % ce == 0 and dent % 128 == 0

    def body(srdr, es0, es1, es2, es3, ed0, ed1, ed2, ed3, hA, hB,
             accA, accB, d0, d1, d2, d3, *scr):
        sets = [scr[6 * b:6 * b + 6] for b in range(nbuf)]
        acc_sh, den0_sh, den1_sh = scr[6 * nbuf:6 * nbuf + 3]
        gsems = scr[6 * nbuf + 3:6 * nbuf + 3 + nbuf]
        ssems = scr[6 * nbuf + 3 + nbuf:6 * nbuf + 3 + 2 * nbuf]
        c = lax.axis_index("c")
        s = lax.axis_index("s")
        zero16 = lax.broadcast(jnp.float32(0), (16,))

        idx0, esa0, eda0, h0, w00, w10 = sets[0]

        def zr_body(r, _):
            h0[r, pl.ds(0, 16)] = zero16
            h0[r, pl.ds(16, 16)] = zero16
            return 0
        lax.fori_loop(0, ce, zr_body, 0)

        def zd_body(j, _):
            w00[pl.ds(j * 16, 16)] = zero16
            return 0
        lax.fori_loop(0, ce // 16, zd_body, 0)

        for t in range(nrt // ce):
            pltpu.sync_copy(h0, acc_sh.at[pl.ds(s * nrt + t * ce, ce), :])
        for t in range(dent // 128):
            pltpu.sync_copy(w00, den0_sh.at[pl.ds(s * dent + t * 128, 128)])
            pltpu.sync_copy(w00, den1_sh.at[pl.ds(s * dent + t * 128, 128)])
        plsc.subcore_barrier()

        def g_copies(b, m, tes_a, tes_b, ted_a, ted_b, t_h):
            idx, esa, eda, h_b, w0_b, w1_b = sets[b]
            return [(tes_a.at[idx.at[0]], esa, gsems[b]),
                    (tes_b.at[idx.at[0]], w1_b, gsems[b]),
                    (ted_a.at[idx.at[1]], eda, gsems[b]),
                    (ted_b.at[idx.at[1]], w0_b, gsems[b]),
                    (t_h.at[idx.at[0]], h_b, gsems[b])]

        def fire_g(b, m):
            idx = sets[b][0]
            pltpu.sync_copy(srdr.at[m * ns + s], idx)

            @pl.when(c == 0)
            def _():
                for src_, dst_, sem_ in g_copies(b, m, es0, es1, ed0, ed1, hA):
                    pltpu.async_copy(src_, dst_, sem_)

            @pl.when(c == 1)
            def _():
                for src_, dst_, sem_ in g_copies(b, m, es2, es3, ed2, ed3, hB):
                    pltpu.async_copy(src_, dst_, sem_)

        def drain_g(b, m):
            for src_, dst_, sem_ in g_copies(b, m, es0, es1, ed0, ed1, hA):
                pltpu.make_async_copy(src_, dst_, sem_).wait()

        def s_copies(b):
            idx, esa, eda, h_b, w0_b, w1_b = sets[b]
            return [(h_b, acc_sh.at[idx.at[1]], ssems[b]),
                    (esa, den0_sh.at[idx.at[1]], ssems[b]),
                    (eda, den1_sh.at[idx.at[1]], ssems[b])]

        def fire_s(b):
            for src_, dst_, sem_ in s_copies(b):
                pltpu.async_copy(src_, dst_, sem_, add=True)

        def drain_s(b):
            for src_, dst_, sem_ in s_copies(b):
                pltpu.make_async_copy(src_, dst_, sem_).wait()

        def compute(b, m):
            idx, esa, eda, h_b, w0_b, w1_b = sets[b]
            base_e = (m * ns + s) * ce

            def g_body(g, _):
                sl = pl.ds(g * 16, 16)
                e0 = esa[sl] + eda[sl]
                e0 = jnp.where(e0 > 0, e0, 0.2 * e0)
                wv0 = jnp.exp(e0)
                e1 = w1_b[sl] + w0_b[sl]
                e1 = jnp.where(e1 > 0, e1, 0.2 * e1)
                wv1 = jnp.exp(e1)
                ge = base_e + g * 16 + lax.iota(jnp.int32, 16)
                msk = ge < e_real
                wv0 = jnp.where(msk, wv0, 0.0)
                wv1 = jnp.where(msk, wv1, 0.0)
                esa[sl] = wv0
                eda[sl] = wv1
                for i in range(16):
                    e_i = g * 16 + i
                    s0 = lax.broadcast(wv0[i], (16,))
                    s1 = lax.broadcast(wv1[i], (16,))
                    h_b[e_i, pl.ds(0, 16)] = s0 * h_b[e_i, pl.ds(0, 16)]
                    h_b[e_i, pl.ds(16, 16)] = s1 * h_b[e_i, pl.ds(16, 16)]
                return 0
            lax.fori_loop(0, ce // 16, g_body, 0)

        fire_g(0, 0)
        fire_g(1, 1)

        def loop_body(t, _):
            for b in range(nbuf):
                m = nbuf * t + b
                drain_g(b, m)
                compute(b, m)
                fire_s(b)
                bp = (b + 2) % nbuf

                @pl.when(m >= 2)
                def _():
                    drain_s(bp)

                @pl.when(m < chunks - 2)
                def _():
                    fire_g(bp, m + 2)
            return 0

        lax.fori_loop(0, chunks // nbuf, loop_body, 0)
        drain_s(2)
        drain_s(3)
        plsc.subcore_barrier()

        @pl.when(c == 0)
        def _():
            pltpu.sync_copy(acc_sh.at[pl.ds(s * nrt, nrt), :], accA.at[pl.ds(s * nrt, nrt), :])
            pltpu.sync_copy(den0_sh.at[pl.ds(s * dent, dent)], d0.at[pl.ds(s * dent, dent)])
            pltpu.sync_copy(den1_sh.at[pl.ds(s * dent, dent)], d1.at[pl.ds(s * dent, dent)])

        @pl.when(c == 1)
        def _():
            pltpu.sync_copy(acc_sh.at[pl.ds(s * nrt, nrt), :], accB.at[pl.ds(s * nrt, nrt), :])
            pltpu.sync_copy(den0_sh.at[pl.ds(s * dent, dent)], d2.at[pl.ds(s * dent, dent)])
            pltpu.sync_copy(den1_sh.at[pl.ds(s * dent, dent)], d3.at[pl.ds(s * dent, dent)])

    f32 = jnp.float32
    out_type = [jax.ShapeDtypeStruct((npad_acc, 32), f32),
                jax.ShapeDtypeStruct((npad_acc, 32), f32)] + \
               [jax.ShapeDtypeStruct((npad_den,), f32)] * 4
    per_set = [
        pltpu.VMEM((2, 128), jnp.int32),
        pltpu.VMEM((ce,), f32),
        pltpu.VMEM((ce,), f32),
        pltpu.VMEM((ce, 32), f32),
        pltpu.VMEM((ce,), f32),
        pltpu.VMEM((ce,), f32),
    ]
    scratch_types = per_set * 4 + [
        pltpu.VMEM_SHARED((npad_acc, 32), f32),
        pltpu.VMEM_SHARED((npad_den,), f32),
        pltpu.VMEM_SHARED((npad_den,), f32),
    ] + [pltpu.SemaphoreType.DMA] * 8
    mesh = plsc.VectorSubcoreMesh(core_axis_name="c", subcore_axis_name="s",
                                  num_cores=2, num_subcores=ns)
    return pl.kernel(body, out_type=out_type, mesh=mesh,
                     scratch_types=scratch_types,
                     compiler_params=pltpu.CompilerParams(use_tc_tiling_on_sc=False))


_EDGE_ROWS_PAD = 6272
_CHUNKS = 392
_NPAD_ACC = 51200
_NPAD_DEN = 51200


def _edges(hA, hB, es4, ed4, srdr):
    fn = _build_edge_kernel(N, E, _EDGE_ROWS_PAD, _CHUNKS, _NPAD_ACC, _NPAD_DEN)
    es = [es4[:, k] for k in range(HEADS)]
    ed = [ed4[:, k] for k in range(HEADS)]
    accA, accB, d0, d1, d2, d3 = fn(srdr, es[0], es[1], es[2], es[3],
                                    ed[0], ed[1], ed[2], ed[3], hA, hB)
    return accA[:N], accB[:N], [d[:N] for d in (d0, d1, d2, d3)]


def kernel(x, edge_index, fp_W, fp_b, fp_g, fp_beta, ip_W, ip_b,
           gat_W, gat_asrc, gat_adst, gat_bias, proj_W, proj_b, ln_g, ln_b):
    pad = _EDGE_ROWS_PAD * 128 - E
    zpad = jnp.zeros((pad,), jnp.int32)
    srcr = jnp.concatenate([edge_index[0], zpad]).reshape(_EDGE_ROWS_PAD, 1, 128)
    dstr = jnp.concatenate([edge_index[1], zpad]).reshape(_EDGE_ROWS_PAD, 1, 128)
    srdr = jnp.concatenate([srcr, dstr], axis=1)
    Wc = [gat_W[l].transpose(1, 0, 2).reshape(HID, HEADS * HD) for l in range(LAYERS)]
    asc = [gat_asrc[l].reshape(HEADS * HD) for l in range(LAYERS)]
    adc = [gat_adst[l].reshape(HEADS * HD) for l in range(LAYERS)]
    bc = [gat_bias[l].reshape(HEADS * HD) for l in range(LAYERS)]

    h, hA, hB, es4, ed4 = _dense_in(x, fp_W, fp_b, fp_g, fp_beta, ip_W, ip_b,
                                    Wc[0], asc[0], adc[0])
    for l in range(LAYERS):
        accA, accB, d4 = _edges(hA, hB, es4, ed4, srdr)
        last = l == LAYERS - 1
        nxt = l + 1 if not last else l
        res = _dense_layer(h, accA, accB, d4, bc[l], proj_W[l], proj_b[l],
                           ln_g[l], ln_b[l], Wc[nxt], asc[nxt], adc[nxt], last)
        if last:
            (h,) = res
        else:
            h, hA, hB, es4, ed4 = res
    return h


# speedup vs baseline: 118.4738x; 1.1679x over previous
import functools

import jax
import jax.numpy as jnp
from jax import lax
from jax.experimental import pallas as pl
from jax.experimental.pallas import tpu as pltpu
from jax.experimental.pallas import tpu_sc as plsc

N = 50000
E = 800000
D_IN = 3
HID = 64
HEADS = 4
HD = 16
LAYERS = 3

NPAD = 51200
BLK = 2048
GRID = NPAD // BLK

_INTERP = False


def _ln(h, g, b):
    m = jnp.mean(h, axis=-1, keepdims=True)
    v = jnp.mean((h - m) ** 2, axis=-1, keepdims=True)
    return (h - m) * jax.lax.rsqrt(v + 1e-5) * g + b


def _head_logits(hp, acat):
    t = hp * acat
    return [jnp.sum(t[:, k * HD:(k + 1) * HD], axis=1) for k in range(HEADS)]


def _dense_in_body(x_ref, fpW, fpb, fpg, fpbeta, ipW, ipb, W0, as0, ad0,
                   h_out, hA, hB, *esed):
    x = x_ref[...]
    h = jnp.dot(x, fpW[...], preferred_element_type=jnp.float32) + fpb[...]
    h = _ln(h, fpg[...], fpbeta[...])
    h = jnp.where(h > 0, h, jnp.exp(jnp.minimum(h, 0.0)) - 1.0)
    h = jnp.dot(h, ipW[...], preferred_element_type=jnp.float32) + ipb[...]
    h_out[...] = h
    hp = jnp.dot(h, W0[...], preferred_element_type=jnp.float32)
    hA[...] = hp[:, :32]
    hB[...] = hp[:, 32:]
    for r, v in zip(esed[:4], _head_logits(hp, as0[...])):
        r[...] = v
    for r, v in zip(esed[4:], _head_logits(hp, ad0[...])):
        r[...] = v


def _dense_layer_body(h_ref, accA, accB, bias, projW, projb,
                      lng, lnb, Wn, asn, adn,
                      h_out, hA=None, hB=None, *esed, last):
    mh = jnp.concatenate([accA[...], accB[...]], axis=1) + bias[...]
    out = jnp.dot(mh, projW[...], preferred_element_type=jnp.float32) + projb[...]
    h = _ln(out + h_ref[...], lng[...], lnb[...])
    h_out[...] = h
    if not last:
        hp = jnp.dot(h, Wn[...], preferred_element_type=jnp.float32)
        hA[...] = hp[:, :32]
        hB[...] = hp[:, 32:]
        for r, v in zip(esed[:4], _head_logits(hp, asn[...])):
            r[...] = v
        for r, v in zip(esed[4:], _head_logits(hp, adn[...])):
            r[...] = v


def _full(shape):
    return pl.BlockSpec(shape, lambda i: tuple(0 for _ in shape))


def _rows(width):
    return pl.BlockSpec((BLK, width), lambda i: (i, 0))


def _rows1():
    return pl.BlockSpec((BLK,), lambda i: (i,))


def _dense_in(x, fpW, fpb, fpg, fpbeta, ipW, ipb, W0, as0, ad0):
    out_shapes = [
        jax.ShapeDtypeStruct((NPAD, HID), jnp.float32),
        jax.ShapeDtypeStruct((NPAD, 32), jnp.float32),
        jax.ShapeDtypeStruct((NPAD, 32), jnp.float32),
    ] + [jax.ShapeDtypeStruct((NPAD,), jnp.float32)] * 8
    return pl.pallas_call(
        _dense_in_body,
        grid=(GRID,),
        in_specs=[_rows(D_IN), _full((D_IN, HID)), _full((HID,)), _full((HID,)),
                  _full((HID,)), _full((HID, HID)), _full((HID,)),
                  _full((HID, HID)), _full((HID,)), _full((HID,))],
        out_specs=[_rows(HID), _rows(32), _rows(32)] + [_rows1()] * 8,
        out_shape=out_shapes,
        interpret=_INTERP,
    )(x, fpW, fpb, fpg, fpbeta, ipW, ipb, W0, as0, ad0)


def _dense_layer(h, accA, accB, bias, projW, projb, lng, lnb, Wn, asn, adn, last):
    out_shapes = [jax.ShapeDtypeStruct((NPAD, HID), jnp.float32)]
    out_specs = [_rows(HID)]
    if not last:
        out_shapes += [
            jax.ShapeDtypeStruct((NPAD, 32), jnp.float32),
            jax.ShapeDtypeStruct((NPAD, 32), jnp.float32),
        ] + [jax.ShapeDtypeStruct((NPAD,), jnp.float32)] * 8
        out_specs += [_rows(32), _rows(32)] + [_rows1()] * 8
    return pl.pallas_call(
        functools.partial(_dense_layer_body, last=last),
        grid=(GRID,),
        in_specs=[_rows(HID), _rows(32), _rows(32),
                  _full((HID,)), _full((HID, HID)), _full((HID,)),
                  _full((HID,)), _full((HID,)),
                  _full((HID, HID)), _full((HID,)), _full((HID,))],
        out_specs=out_specs,
        out_shape=out_shapes,
        interpret=_INTERP,
    )(h, accA, accB, bias, projW, projb, lng, lnb, Wn, asn, adn)


def _build_edge_kernel(n, e_real, rows_pad, chunks, npad_acc, npad_den):
    ns = 16
    ce = 128
    nbuf = 4
    nrt = npad_acc // ns
    dent = npad_den // ns
    assert chunks % nbuf == 0 and nrt % ce == 0 and dent % 128 == 0

    def body(srdr, es0, es1, es2, es3, ed0, ed1, ed2, ed3, hA, hB,
             accA, accB, *scr):
        sets = [scr[6 * b:6 * b + 6] for b in range(nbuf)]
        acc_sh, den0_sh, den1_sh = scr[6 * nbuf:6 * nbuf + 3]
        gsems = scr[6 * nbuf + 3:6 * nbuf + 3 + nbuf]
        ssems = scr[6 * nbuf + 3 + nbuf:6 * nbuf + 3 + 2 * nbuf]
        c = lax.axis_index("c")
        s = lax.axis_index("s")
        zero16 = lax.broadcast(jnp.float32(0), (16,))

        idx0, esa0, eda0, h0, w00, w10 = sets[0]

        def zr_body(r, _):
            h0[r, pl.ds(0, 16)] = zero16
            h0[r, pl.ds(16, 16)] = zero16
            return 0
        lax.fori_loop(0, ce, zr_body, 0)

        def zd_body(j, _):
            w00[pl.ds(j * 16, 16)] = zero16
            return 0
        lax.fori_loop(0, ce // 16, zd_body, 0)

        for t in range(nrt // ce):
            pltpu.sync_copy(h0, acc_sh.at[pl.ds(s * nrt + t * ce, ce), :])
        for t in range(dent // 128):
            pltpu.sync_copy(w00, den0_sh.at[pl.ds(s * dent + t * 128, 128)])
            pltpu.sync_copy(w00, den1_sh.at[pl.ds(s * dent + t * 128, 128)])
        plsc.subcore_barrier()

        def g_copies(b, m, tes_a, tes_b, ted_a, ted_b, t_h):
            idx, esa, eda, h_b, w0_b, w1_b = sets[b]
            return [(tes_a.at[idx.at[0]], esa, gsems[b]),
                    (tes_b.at[idx.at[0]], w1_b, gsems[b]),
                    (ted_a.at[idx.at[1]], eda, gsems[b]),
                    (ted_b.at[idx.at[1]], w0_b, gsems[b]),
                    (t_h.at[idx.at[0]], h_b, gsems[b])]

        def fire_g(b, m):
            idx = sets[b][0]
            pltpu.sync_copy(srdr.at[m * ns + s], idx)

            @pl.when(c == 0)
            def _():
                for src_, dst_, sem_ in g_copies(b, m, es0, es1, ed0, ed1, hA):
                    pltpu.async_copy(src_, dst_, sem_)

            @pl.when(c == 1)
            def _():
                for src_, dst_, sem_ in g_copies(b, m, es2, es3, ed2, ed3, hB):
                    pltpu.async_copy(src_, dst_, sem_)

        def drain_g(b, m):
            for src_, dst_, sem_ in g_copies(b, m, es0, es1, ed0, ed1, hA):
                pltpu.make_async_copy(src_, dst_, sem_).wait()

        def s_copies(b):
            idx, esa, eda, h_b, w0_b, w1_b = sets[b]
            return [(h_b, acc_sh.at[idx.at[1]], ssems[b]),
                    (esa, den0_sh.at[idx.at[1]], ssems[b]),
                    (eda, den1_sh.at[idx.at[1]], ssems[b])]

        def fire_s(b):
            for src_, dst_, sem_ in s_copies(b):
                pltpu.async_copy(src_, dst_, sem_, add=True)

        def drain_s(b):
            for src_, dst_, sem_ in s_copies(b):
                pltpu.make_async_copy(src_, dst_, sem_).wait()

        def compute(b, m):
            idx, esa, eda, h_b, w0_b, w1_b = sets[b]
            base_e = (m * ns + s) * ce

            def g_body(g, _):
                sl = pl.ds(g * 16, 16)
                e0 = esa[sl] + eda[sl]
                e0 = jnp.where(e0 > 0, e0, 0.2 * e0)
                wv0 = jnp.exp(e0)
                e1 = w1_b[sl] + w0_b[sl]
                e1 = jnp.where(e1 > 0, e1, 0.2 * e1)
                wv1 = jnp.exp(e1)
                ge = base_e + g * 16 + lax.iota(jnp.int32, 16)
                msk = ge < e_real
                wv0 = jnp.where(msk, wv0, 0.0)
                wv1 = jnp.where(msk, wv1, 0.0)
                esa[sl] = wv0
                eda[sl] = wv1
                for i in range(16):
                    e_i = g * 16 + i
                    s0 = lax.broadcast(wv0[i], (16,))
                    s1 = lax.broadcast(wv1[i], (16,))
                    h_b[e_i, pl.ds(0, 16)] = s0 * h_b[e_i, pl.ds(0, 16)]
                    h_b[e_i, pl.ds(16, 16)] = s1 * h_b[e_i, pl.ds(16, 16)]
                return 0
            lax.fori_loop(0, ce // 16, g_body, 0)

        fire_g(0, 0)
        fire_g(1, 1)

        def loop_body(t, _):
            for b in range(nbuf):
                m = nbuf * t + b
                drain_g(b, m)
                compute(b, m)
                fire_s(b)
                bp = (b + 2) % nbuf

                @pl.when(m >= 2)
                def _():
                    drain_s(bp)

                @pl.when(m < chunks - 2)
                def _():
                    fire_g(bp, m + 2)
            return 0

        lax.fori_loop(0, chunks // nbuf, loop_body, 0)
        drain_s(2)
        drain_s(3)
        plsc.subcore_barrier()

        def co_body(t, _):
            off = s * nrt + t * ce
            pltpu.sync_copy(acc_sh.at[pl.ds(off, ce), :], h0)
            pltpu.sync_copy(den0_sh.at[pl.ds(off, ce)], esa0)
            pltpu.sync_copy(den1_sh.at[pl.ds(off, ce)], eda0)

            def dg_body(g, _):
                dr0 = 1.0 / (esa0[pl.ds(g * 16, 16)] + 1e-16)
                dr1 = 1.0 / (eda0[pl.ds(g * 16, 16)] + 1e-16)
                for i in range(16):
                    r = g * 16 + i
                    h0[r, pl.ds(0, 16)] = lax.broadcast(dr0[i], (16,)) * h0[r, pl.ds(0, 16)]
                    h0[r, pl.ds(16, 16)] = lax.broadcast(dr1[i], (16,)) * h0[r, pl.ds(16, 16)]
                return 0
            lax.fori_loop(0, ce // 16, dg_body, 0)

            @pl.when(c == 0)
            def _():
                pltpu.sync_copy(h0, accA.at[pl.ds(off, ce), :])

            @pl.when(c == 1)
            def _():
                pltpu.sync_copy(h0, accB.at[pl.ds(off, ce), :])
            return 0

        lax.fori_loop(0, nrt // ce, co_body, 0)

    f32 = jnp.float32
    out_type = [jax.ShapeDtypeStruct((npad_acc, 32), f32),
                jax.ShapeDtypeStruct((npad_acc, 32), f32)]
    per_set = [
        pltpu.VMEM((2, 128), jnp.int32),
        pltpu.VMEM((ce,), f32),
        pltpu.VMEM((ce,), f32),
        pltpu.VMEM((ce, 32), f32),
        pltpu.VMEM((ce,), f32),
        pltpu.VMEM((ce,), f32),
    ]
    scratch_types = per_set * 4 + [
        pltpu.VMEM_SHARED((npad_acc, 32), f32),
        pltpu.VMEM_SHARED((npad_den,), f32),
        pltpu.VMEM_SHARED((npad_den,), f32),
    ] + [pltpu.SemaphoreType.DMA] * 8
    mesh = plsc.VectorSubcoreMesh(core_axis_name="c", subcore_axis_name="s",
                                  num_cores=2, num_subcores=ns)
    return pl.kernel(body, out_type=out_type, mesh=mesh,
                     scratch_types=scratch_types,
                     compiler_params=pltpu.CompilerParams(use_tc_tiling_on_sc=False))


_EDGE_ROWS_PAD = 6272
_CHUNKS = 392
_NPAD_ACC = 51200
_NPAD_DEN = 51200


def _edges(hA, hB, es, ed, srdr):
    fn = _build_edge_kernel(NPAD, E, _EDGE_ROWS_PAD, _CHUNKS, _NPAD_ACC, _NPAD_DEN)
    return fn(srdr, es[0], es[1], es[2], es[3], ed[0], ed[1], ed[2], ed[3],
              hA, hB)


def kernel(x, edge_index, fp_W, fp_b, fp_g, fp_beta, ip_W, ip_b,
           gat_W, gat_asrc, gat_adst, gat_bias, proj_W, proj_b, ln_g, ln_b):
    pad = _EDGE_ROWS_PAD * 128 - E
    zpad = jnp.zeros((pad,), jnp.int32)
    srcr = jnp.concatenate([edge_index[0], zpad]).reshape(_EDGE_ROWS_PAD, 1, 128)
    dstr = jnp.concatenate([edge_index[1], zpad]).reshape(_EDGE_ROWS_PAD, 1, 128)
    srdr = jnp.concatenate([srcr, dstr], axis=1)
    Wc = [gat_W[l].transpose(1, 0, 2).reshape(HID, HEADS * HD) for l in range(LAYERS)]
    asc = [gat_asrc[l].reshape(HEADS * HD) for l in range(LAYERS)]
    adc = [gat_adst[l].reshape(HEADS * HD) for l in range(LAYERS)]
    bc = [gat_bias[l].reshape(HEADS * HD) for l in range(LAYERS)]

    x_pad = jnp.zeros((NPAD, D_IN), jnp.float32).at[:N].set(x)
    res = _dense_in(x_pad, fp_W, fp_b, fp_g, fp_beta, ip_W, ip_b,
                    Wc[0], asc[0], adc[0])
    h, hA, hB = res[0], res[1], res[2]
    es, ed = res[3:7], res[7:11]
    for l in range(LAYERS):
        accA, accB = _edges(hA, hB, es, ed, srdr)
        last = l == LAYERS - 1
        nxt = l + 1 if not last else l
        res = _dense_layer(h, accA, accB, bc[l], proj_W[l], proj_b[l],
                           ln_g[l], ln_b[l], Wc[nxt], asc[nxt], adc[nxt], last)
        h = res[0]
        if not last:
            hA, hB = res[1], res[2]
            es, ed = res[3:7], res[7:11]
    return h[:N]


# speedup vs baseline: 135.3789x; 1.1427x over previous
import functools

import jax
import jax.numpy as jnp
from jax import lax
from jax.experimental import pallas as pl
from jax.experimental.pallas import tpu as pltpu
from jax.experimental.pallas import tpu_sc as plsc

N = 50000
E = 800000
D_IN = 3
HID = 64
HEADS = 4
HD = 16
LAYERS = 3

NPAD = 51200
BLK = 2048
GRID = NPAD // BLK

_INTERP = False


def _ln(h, g, b):
    m = jnp.mean(h, axis=-1, keepdims=True)
    v = jnp.mean((h - m) ** 2, axis=-1, keepdims=True)
    return (h - m) * jax.lax.rsqrt(v + 1e-5) * g + b


def _head_logits(hp, amat):
    e4 = jnp.dot(hp, amat, preferred_element_type=jnp.float32)
    return [e4[:, k] for k in range(HEADS)]


def _dense_in_body(x_ref, fpW, fpb, fpg, fpbeta, ipW, ipb, W0, as0, ad0,
                   h_out, hA, hB, *esed):
    x = x_ref[...]
    h = jnp.dot(x, fpW[...], preferred_element_type=jnp.float32) + fpb[...]
    h = _ln(h, fpg[...], fpbeta[...])
    h = jnp.where(h > 0, h, jnp.exp(jnp.minimum(h, 0.0)) - 1.0)
    h = jnp.dot(h, ipW[...], preferred_element_type=jnp.float32) + ipb[...]
    h_out[...] = h
    hp = jnp.dot(h, W0[...], preferred_element_type=jnp.float32)
    hA[...] = hp[:, :32]
    hB[...] = hp[:, 32:]
    for r, v in zip(esed[:4], _head_logits(hp, as0[...])):
        r[...] = v
    for r, v in zip(esed[4:], _head_logits(hp, ad0[...])):
        r[...] = v


def _dense_layer_body(h_ref, accA, accB, bias, projW, projb,
                      lng, lnb, Wn, asn, adn,
                      h_out, hA=None, hB=None, *esed, last):
    mh = jnp.concatenate([accA[...], accB[...]], axis=1) + bias[...]
    out = jnp.dot(mh, projW[...], preferred_element_type=jnp.float32) + projb[...]
    h = _ln(out + h_ref[...], lng[...], lnb[...])
    h_out[...] = h
    if not last:
        hp = jnp.dot(h, Wn[...], preferred_element_type=jnp.float32)
        hA[...] = hp[:, :32]
        hB[...] = hp[:, 32:]
        for r, v in zip(esed[:4], _head_logits(hp, asn[...])):
            r[...] = v
        for r, v in zip(esed[4:], _head_logits(hp, adn[...])):
            r[...] = v


def _full(shape):
    return pl.BlockSpec(shape, lambda i: tuple(0 for _ in shape))


def _rows(width):
    return pl.BlockSpec((BLK, width), lambda i: (i, 0))


def _rows1():
    return pl.BlockSpec((BLK,), lambda i: (i,))


def _dense_in(x, fpW, fpb, fpg, fpbeta, ipW, ipb, W0, as0, ad0):
    out_shapes = [
        jax.ShapeDtypeStruct((NPAD, HID), jnp.float32),
        jax.ShapeDtypeStruct((NPAD, 32), jnp.float32),
        jax.ShapeDtypeStruct((NPAD, 32), jnp.float32),
    ] + [jax.ShapeDtypeStruct((NPAD,), jnp.float32)] * 8
    return pl.pallas_call(
        _dense_in_body,
        grid=(GRID,),
        in_specs=[_rows(D_IN), _full((D_IN, HID)), _full((HID,)), _full((HID,)),
                  _full((HID,)), _full((HID, HID)), _full((HID,)),
                  _full((HID, HID)), _full((HID, HEADS)), _full((HID, HEADS))],
        out_specs=[_rows(HID), _rows(32), _rows(32)] + [_rows1()] * 8,
        out_shape=out_shapes,
        interpret=_INTERP,
    )(x, fpW, fpb, fpg, fpbeta, ipW, ipb, W0, as0, ad0)


def _dense_layer(h, accA, accB, bias, projW, projb, lng, lnb, Wn, asn, adn, last):
    out_shapes = [jax.ShapeDtypeStruct((NPAD, HID), jnp.float32)]
    out_specs = [_rows(HID)]
    if not last:
        out_shapes += [
            jax.ShapeDtypeStruct((NPAD, 32), jnp.float32),
            jax.ShapeDtypeStruct((NPAD, 32), jnp.float32),
        ] + [jax.ShapeDtypeStruct((NPAD,), jnp.float32)] * 8
        out_specs += [_rows(32), _rows(32)] + [_rows1()] * 8
    return pl.pallas_call(
        functools.partial(_dense_layer_body, last=last),
        grid=(GRID,),
        in_specs=[_rows(HID), _rows(32), _rows(32),
                  _full((HID,)), _full((HID, HID)), _full((HID,)),
                  _full((HID,)), _full((HID,)),
                  _full((HID, HID)), _full((HID, HEADS)), _full((HID, HEADS))],
        out_specs=out_specs,
        out_shape=out_shapes,
        interpret=_INTERP,
    )(h, accA, accB, bias, projW, projb, lng, lnb, Wn, asn, adn)


def _build_edge_kernel(n, e_real, rows_pad, chunks, npad_acc, npad_den):
    ns = 16
    ce = 128
    nbuf = 4
    nrt = npad_acc // ns
    dent = npad_den // ns
    assert chunks % nbuf == 0 and nrt % ce == 0 and dent % 128 == 0

    def body(srdr, es0, es1, es2, es3, ed0, ed1, ed2, ed3, hA, hB,
             accA, accB, *scr):
        sets = [scr[6 * b:6 * b + 6] for b in range(nbuf)]
        acc_sh, den0_sh, den1_sh = scr[6 * nbuf:6 * nbuf + 3]
        gsems = scr[6 * nbuf + 3:6 * nbuf + 3 + nbuf]
        ssems = scr[6 * nbuf + 3 + nbuf:6 * nbuf + 3 + 2 * nbuf]
        c = lax.axis_index("c")
        s = lax.axis_index("s")
        zero16 = lax.broadcast(jnp.float32(0), (16,))

        idx0, esa0, eda0, h0, w00, w10 = sets[0]

        def zr_body(r, _):
            h0[r, pl.ds(0, 16)] = zero16
            h0[r, pl.ds(16, 16)] = zero16
            return 0
        lax.fori_loop(0, ce, zr_body, 0)

        def zd_body(j, _):
            w00[pl.ds(j * 16, 16)] = zero16
            return 0
        lax.fori_loop(0, ce // 16, zd_body, 0)

        for t in range(nrt // ce):
            pltpu.sync_copy(h0, acc_sh.at[pl.ds(s * nrt + t * ce, ce), :])
        for t in range(dent // 128):
            pltpu.sync_copy(w00, den0_sh.at[pl.ds(s * dent + t * 128, 128)])
            pltpu.sync_copy(w00, den1_sh.at[pl.ds(s * dent + t * 128, 128)])
        plsc.subcore_barrier()

        def g_copies(b, m, tes_a, tes_b, ted_a, ted_b, t_h):
            idx, esa, eda, h_b, w0_b, w1_b = sets[b]
            return [(tes_a.at[idx.at[0]], esa, gsems[b]),
                    (tes_b.at[idx.at[0]], w1_b, gsems[b]),
                    (ted_a.at[idx.at[1]], eda, gsems[b]),
                    (ted_b.at[idx.at[1]], w0_b, gsems[b]),
                    (t_h.at[idx.at[0]], h_b, gsems[b])]

        def fire_g(b, m):
            idx = sets[b][0]
            pltpu.sync_copy(srdr.at[m * ns + s], idx)

            @pl.when(c == 0)
            def _():
                for src_, dst_, sem_ in g_copies(b, m, es0, es1, ed0, ed1, hA):
                    pltpu.async_copy(src_, dst_, sem_)

            @pl.when(c == 1)
            def _():
                for src_, dst_, sem_ in g_copies(b, m, es2, es3, ed2, ed3, hB):
                    pltpu.async_copy(src_, dst_, sem_)

        def drain_g(b, m):
            for src_, dst_, sem_ in g_copies(b, m, es0, es1, ed0, ed1, hA):
                pltpu.make_async_copy(src_, dst_, sem_).wait()

        def s_copies(b):
            idx, esa, eda, h_b, w0_b, w1_b = sets[b]
            return [(h_b, acc_sh.at[idx.at[1]], ssems[b]),
                    (esa, den0_sh.at[idx.at[1]], ssems[b]),
                    (eda, den1_sh.at[idx.at[1]], ssems[b])]

        def fire_s(b):
            for src_, dst_, sem_ in s_copies(b):
                pltpu.async_copy(src_, dst_, sem_, add=True)

        def drain_s(b):
            for src_, dst_, sem_ in s_copies(b):
                pltpu.make_async_copy(src_, dst_, sem_).wait()

        def compute(b, m):
            idx, esa, eda, h_b, w0_b, w1_b = sets[b]
            base_e = (m * ns + s) * ce

            def g_body(g, _):
                sl = pl.ds(g * 16, 16)
                e0 = esa[sl] + eda[sl]
                e0 = jnp.where(e0 > 0, e0, 0.2 * e0)
                wv0 = jnp.exp(e0)
                e1 = w1_b[sl] + w0_b[sl]
                e1 = jnp.where(e1 > 0, e1, 0.2 * e1)
                wv1 = jnp.exp(e1)
                ge = base_e + g * 16 + lax.iota(jnp.int32, 16)
                msk = ge < e_real
                wv0 = jnp.where(msk, wv0, 0.0)
                wv1 = jnp.where(msk, wv1, 0.0)
                esa[sl] = wv0
                eda[sl] = wv1
                for i in range(16):
                    e_i = g * 16 + i
                    s0 = lax.broadcast(wv0[i], (16,))
                    s1 = lax.broadcast(wv1[i], (16,))
                    h_b[e_i, pl.ds(0, 16)] = s0 * h_b[e_i, pl.ds(0, 16)]
                    h_b[e_i, pl.ds(16, 16)] = s1 * h_b[e_i, pl.ds(16, 16)]
                return 0
            lax.fori_loop(0, ce // 16, g_body, 0)

        fire_g(0, 0)
        fire_g(1, 1)

        def loop_body(t, _):
            for b in range(nbuf):
                m = nbuf * t + b
                drain_g(b, m)
                compute(b, m)
                fire_s(b)
                bp = (b + 2) % nbuf

                @pl.when(m >= 2)
                def _():
                    drain_s(bp)

                @pl.when(m < chunks - 2)
                def _():
                    fire_g(bp, m + 2)
            return 0

        lax.fori_loop(0, chunks // nbuf, loop_body, 0)
        drain_s(2)
        drain_s(3)
        plsc.subcore_barrier()

        def co_body(t, _):
            off = s * nrt + t * ce
            pltpu.sync_copy(acc_sh.at[pl.ds(off, ce), :], h0)
            pltpu.sync_copy(den0_sh.at[pl.ds(off, ce)], esa0)
            pltpu.sync_copy(den1_sh.at[pl.ds(off, ce)], eda0)

            def dg_body(g, _):
                dr0 = 1.0 / (esa0[pl.ds(g * 16, 16)] + 1e-16)
                dr1 = 1.0 / (eda0[pl.ds(g * 16, 16)] + 1e-16)
                for i in range(16):
                    r = g * 16 + i
                    h0[r, pl.ds(0, 16)] = lax.broadcast(dr0[i], (16,)) * h0[r, pl.ds(0, 16)]
                    h0[r, pl.ds(16, 16)] = lax.broadcast(dr1[i], (16,)) * h0[r, pl.ds(16, 16)]
                return 0
            lax.fori_loop(0, ce // 16, dg_body, 0)

            @pl.when(c == 0)
            def _():
                pltpu.sync_copy(h0, accA.at[pl.ds(off, ce), :])

            @pl.when(c == 1)
            def _():
                pltpu.sync_copy(h0, accB.at[pl.ds(off, ce), :])
            return 0

        lax.fori_loop(0, nrt // ce, co_body, 0)

    f32 = jnp.float32
    out_type = [jax.ShapeDtypeStruct((npad_acc, 32), f32),
                jax.ShapeDtypeStruct((npad_acc, 32), f32)]
    per_set = [
        pltpu.VMEM((2, 128), jnp.int32),
        pltpu.VMEM((ce,), f32),
        pltpu.VMEM((ce,), f32),
        pltpu.VMEM((ce, 32), f32),
        pltpu.VMEM((ce,), f32),
        pltpu.VMEM((ce,), f32),
    ]
    scratch_types = per_set * 4 + [
        pltpu.VMEM_SHARED((npad_acc, 32), f32),
        pltpu.VMEM_SHARED((npad_den,), f32),
        pltpu.VMEM_SHARED((npad_den,), f32),
    ] + [pltpu.SemaphoreType.DMA] * 8
    mesh = plsc.VectorSubcoreMesh(core_axis_name="c", subcore_axis_name="s",
                                  num_cores=2, num_subcores=ns)
    return pl.kernel(body, out_type=out_type, mesh=mesh,
                     scratch_types=scratch_types,
                     compiler_params=pltpu.CompilerParams(use_tc_tiling_on_sc=False))


_EDGE_ROWS_PAD = 6272
_CHUNKS = 392
_NPAD_ACC = 51200
_NPAD_DEN = 51200


def _edges(hA, hB, es, ed, srdr):
    fn = _build_edge_kernel(NPAD, E, _EDGE_ROWS_PAD, _CHUNKS, _NPAD_ACC, _NPAD_DEN)
    return fn(srdr, es[0], es[1], es[2], es[3], ed[0], ed[1], ed[2], ed[3],
              hA, hB)


def kernel(x, edge_index, fp_W, fp_b, fp_g, fp_beta, ip_W, ip_b,
           gat_W, gat_asrc, gat_adst, gat_bias, proj_W, proj_b, ln_g, ln_b):
    pad = _EDGE_ROWS_PAD * 128 - E
    zpad = jnp.zeros((pad,), jnp.int32)
    srcr = jnp.concatenate([edge_index[0], zpad]).reshape(_EDGE_ROWS_PAD, 1, 128)
    dstr = jnp.concatenate([edge_index[1], zpad]).reshape(_EDGE_ROWS_PAD, 1, 128)
    srdr = jnp.concatenate([srcr, dstr], axis=1)
    Wc = [gat_W[l].transpose(1, 0, 2).reshape(HID, HEADS * HD) for l in range(LAYERS)]
    eye = jnp.repeat(jnp.eye(HEADS, dtype=jnp.float32), HD, axis=0)
    asc = [eye * gat_asrc[l].reshape(HEADS * HD)[:, None] for l in range(LAYERS)]
    adc = [eye * gat_adst[l].reshape(HEADS * HD)[:, None] for l in range(LAYERS)]
    bc = [gat_bias[l].reshape(HEADS * HD) for l in range(LAYERS)]

    x_pad = jnp.zeros((NPAD, D_IN), jnp.float32).at[:N].set(x)
    res = _dense_in(x_pad, fp_W, fp_b, fp_g, fp_beta, ip_W, ip_b,
                    Wc[0], asc[0], adc[0])
    h, hA, hB = res[0], res[1], res[2]
    es, ed = res[3:7], res[7:11]
    for l in range(LAYERS):
        accA, accB = _edges(hA, hB, es, ed, srdr)
        last = l == LAYERS - 1
        nxt = l + 1 if not last else l
        res = _dense_layer(h, accA, accB, bc[l], proj_W[l], proj_b[l],
                           ln_g[l], ln_b[l], Wc[nxt], asc[nxt], adc[nxt], last)
        h = res[0]
        if not last:
            hA, hB = res[1], res[2]
            es, ed = res[3:7], res[7:11]
    return h[:N]


# speedup vs baseline: 152.8234x; 1.1289x over previous
import functools

import jax
import jax.numpy as jnp
from jax import lax
from jax.experimental import pallas as pl
from jax.experimental.pallas import tpu as pltpu
from jax.experimental.pallas import tpu_sc as plsc

N = 50000
E = 800000
D_IN = 3
HID = 64
HEADS = 4
HD = 16
LAYERS = 3

NPAD = 51200
BLK = 2048
GRID = NPAD // BLK

_INTERP = False


def _ln(h, g, b):
    m = jnp.mean(h, axis=-1, keepdims=True)
    v = jnp.mean((h - m) ** 2, axis=-1, keepdims=True)
    return (h - m) * jax.lax.rsqrt(v + 1e-5) * g + b


def _head_logits(hp, amat):
    e4 = jnp.dot(hp, amat, preferred_element_type=jnp.float32)
    return [e4[:, k] for k in range(HEADS)]


def _dense_in_body(x_ref, fpW, fpb, fpg, fpbeta, ipW, ipb, W0, as0, ad0,
                   h_out, hA, hB, *esed):
    x = x_ref[...]
    h = jnp.dot(x, fpW[...], preferred_element_type=jnp.float32) + fpb[...]
    h = _ln(h, fpg[...], fpbeta[...])
    h = jnp.where(h > 0, h, jnp.exp(jnp.minimum(h, 0.0)) - 1.0)
    h = jnp.dot(h, ipW[...], preferred_element_type=jnp.float32) + ipb[...]
    h_out[...] = h
    hp = jnp.dot(h, W0[...], preferred_element_type=jnp.float32)
    hA[...] = hp[:, :32]
    hB[...] = hp[:, 32:]
    for r, v in zip(esed[:4], _head_logits(hp, as0[...])):
        r[...] = v
    for r, v in zip(esed[4:], _head_logits(hp, ad0[...])):
        r[...] = v


def _dense_layer_body(h_ref, accA, accB, bias, projW, projb,
                      lng, lnb, Wn, asn, adn,
                      h_out, hA=None, hB=None, *esed, last):
    mh = jnp.concatenate([accA[...], accB[...]], axis=1) + bias[...]
    out = jnp.dot(mh, projW[...], preferred_element_type=jnp.float32) + projb[...]
    h = _ln(out + h_ref[...], lng[...], lnb[...])
    h_out[...] = h
    if not last:
        hp = jnp.dot(h, Wn[...], preferred_element_type=jnp.float32)
        hA[...] = hp[:, :32]
        hB[...] = hp[:, 32:]
        for r, v in zip(esed[:4], _head_logits(hp, asn[...])):
            r[...] = v
        for r, v in zip(esed[4:], _head_logits(hp, adn[...])):
            r[...] = v


def _full(shape):
    return pl.BlockSpec(shape, lambda i: tuple(0 for _ in shape))


def _rows(width):
    return pl.BlockSpec((BLK, width), lambda i: (i, 0))


def _rows1():
    return pl.BlockSpec((BLK,), lambda i: (i,))


def _dense_in(x, fpW, fpb, fpg, fpbeta, ipW, ipb, W0, as0, ad0):
    out_shapes = [
        jax.ShapeDtypeStruct((NPAD, HID), jnp.float32),
        jax.ShapeDtypeStruct((NPAD, 32), jnp.float32),
        jax.ShapeDtypeStruct((NPAD, 32), jnp.float32),
    ] + [jax.ShapeDtypeStruct((NPAD,), jnp.float32)] * 8
    return pl.pallas_call(
        _dense_in_body,
        grid=(GRID,),
        in_specs=[_rows(D_IN), _full((D_IN, HID)), _full((HID,)), _full((HID,)),
                  _full((HID,)), _full((HID, HID)), _full((HID,)),
                  _full((HID, HID)), _full((HID, HEADS)), _full((HID, HEADS))],
        out_specs=[_rows(HID), _rows(32), _rows(32)] + [_rows1()] * 8,
        out_shape=out_shapes,
        interpret=_INTERP,
    )(x, fpW, fpb, fpg, fpbeta, ipW, ipb, W0, as0, ad0)


def _dense_layer(h, accA, accB, bias, projW, projb, lng, lnb, Wn, asn, adn, last):
    out_shapes = [jax.ShapeDtypeStruct((NPAD, HID), jnp.float32)]
    out_specs = [_rows(HID)]
    if not last:
        out_shapes += [
            jax.ShapeDtypeStruct((NPAD, 32), jnp.float32),
            jax.ShapeDtypeStruct((NPAD, 32), jnp.float32),
        ] + [jax.ShapeDtypeStruct((NPAD,), jnp.float32)] * 8
        out_specs += [_rows(32), _rows(32)] + [_rows1()] * 8
    return pl.pallas_call(
        functools.partial(_dense_layer_body, last=last),
        grid=(GRID,),
        in_specs=[_rows(HID), _rows(32), _rows(32),
                  _full((HID,)), _full((HID, HID)), _full((HID,)),
                  _full((HID,)), _full((HID,)),
                  _full((HID, HID)), _full((HID, HEADS)), _full((HID, HEADS))],
        out_specs=out_specs,
        out_shape=out_shapes,
        interpret=_INTERP,
    )(h, accA, accB, bias, projW, projb, lng, lnb, Wn, asn, adn)


def _build_edge_kernel(n, e_real, rows_pad, chunks, npad_acc, npad_den):
    ns = 16
    ce = 128
    nbuf = 4
    nrt = npad_acc // ns
    dent = npad_den // ns
    assert chunks % nbuf == 0 and nrt % ce == 0 and dent % 128 == 0

    def body(srdr, es0, es1, es2, es3, ed0, ed1, ed2, ed3, hA, hB,
             accA, accB, *scr):
        sets = [scr[7 * b:7 * b + 7] for b in range(nbuf)]
        acc_sh, den0_sh, den1_sh = scr[7 * nbuf:7 * nbuf + 3]
        gsems = scr[7 * nbuf + 3:7 * nbuf + 3 + nbuf]
        ssems = scr[7 * nbuf + 3 + nbuf:7 * nbuf + 3 + 2 * nbuf]
        isems = scr[7 * nbuf + 3 + 2 * nbuf:7 * nbuf + 3 + 3 * nbuf]
        c = lax.axis_index("c")
        s = lax.axis_index("s")
        zero16 = lax.broadcast(jnp.float32(0), (16,))

        idx0, isc0, esa0, eda0, h0, w00, w10 = sets[0]

        def zr_body(r, _):
            h0[r, pl.ds(0, 16)] = zero16
            h0[r, pl.ds(16, 16)] = zero16
            return 0
        lax.fori_loop(0, ce, zr_body, 0)

        def zd_body(j, _):
            w00[pl.ds(j * 16, 16)] = zero16
            return 0
        lax.fori_loop(0, ce // 16, zd_body, 0)

        for t in range(nrt // ce):
            pltpu.sync_copy(h0, acc_sh.at[pl.ds(s * nrt + t * ce, ce), :])
        for t in range(dent // 128):
            pltpu.sync_copy(w00, den0_sh.at[pl.ds(s * dent + t * 128, 128)])
            pltpu.sync_copy(w00, den1_sh.at[pl.ds(s * dent + t * 128, 128)])
        plsc.subcore_barrier()

        def fire_idx(b, m):
            pltpu.async_copy(srdr.at[m * ns + s], sets[b][0], isems[b])

        def g_copies(b, tes_a, tes_b, ted_a, ted_b, t_h):
            idx, isc, esa, eda, h_b, w0_b, w1_b = sets[b]
            return [(tes_a.at[idx.at[0]], esa, gsems[b]),
                    (tes_b.at[idx.at[0]], w1_b, gsems[b]),
                    (ted_a.at[idx.at[1]], eda, gsems[b]),
                    (ted_b.at[idx.at[1]], w0_b, gsems[b]),
                    (t_h.at[idx.at[0]], h_b, gsems[b])]

        def fire_g(b, m):
            pltpu.make_async_copy(srdr.at[m * ns + s], sets[b][0], isems[b]).wait()

            @pl.when(c == 0)
            def _():
                for src_, dst_, sem_ in g_copies(b, es0, es1, ed0, ed1, hA):
                    pltpu.async_copy(src_, dst_, sem_)

            @pl.when(c == 1)
            def _():
                for src_, dst_, sem_ in g_copies(b, es2, es3, ed2, ed3, hB):
                    pltpu.async_copy(src_, dst_, sem_)

        def drain_g(b, m):
            for src_, dst_, sem_ in g_copies(b, es0, es1, ed0, ed1, hA):
                pltpu.make_async_copy(src_, dst_, sem_).wait()

        def s_copies(b):
            idx, isc, esa, eda, h_b, w0_b, w1_b = sets[b]
            return [(h_b, acc_sh.at[isc.at[0]], ssems[b]),
                    (esa, den0_sh.at[isc.at[0]], ssems[b]),
                    (eda, den1_sh.at[isc.at[0]], ssems[b])]

        def fire_s(b):
            for src_, dst_, sem_ in s_copies(b):
                pltpu.async_copy(src_, dst_, sem_, add=True)

        def drain_s(b):
            for src_, dst_, sem_ in s_copies(b):
                pltpu.make_async_copy(src_, dst_, sem_).wait()

        def compute(b, m):
            idx, isc, esa, eda, h_b, w0_b, w1_b = sets[b]
            base_e = (m * ns + s) * ce

            def ic_body(j, _):
                isc[0, pl.ds(j * 16, 16)] = idx[1, pl.ds(j * 16, 16)]
                return 0
            lax.fori_loop(0, ce // 16, ic_body, 0)

            def g_body(g, _):
                sl = pl.ds(g * 16, 16)
                e0 = esa[sl] + eda[sl]
                e0 = jnp.where(e0 > 0, e0, 0.2 * e0)
                wv0 = jnp.exp(e0)
                e1 = w1_b[sl] + w0_b[sl]
                e1 = jnp.where(e1 > 0, e1, 0.2 * e1)
                wv1 = jnp.exp(e1)
                ge = base_e + g * 16 + lax.iota(jnp.int32, 16)
                msk = ge < e_real
                wv0 = jnp.where(msk, wv0, 0.0)
                wv1 = jnp.where(msk, wv1, 0.0)
                esa[sl] = wv0
                eda[sl] = wv1
                for i in range(16):
                    e_i = g * 16 + i
                    s0 = lax.broadcast(wv0[i], (16,))
                    s1 = lax.broadcast(wv1[i], (16,))
                    h_b[e_i, pl.ds(0, 16)] = s0 * h_b[e_i, pl.ds(0, 16)]
                    h_b[e_i, pl.ds(16, 16)] = s1 * h_b[e_i, pl.ds(16, 16)]
                return 0
            lax.fori_loop(0, ce // 16, g_body, 0)

        for b in range(nbuf):
            fire_idx(b, b)
        fire_g(0, 0)
        fire_g(1, 1)

        def loop_body(t, _):
            for b in range(nbuf):
                m = nbuf * t + b
                drain_g(b, m)
                compute(b, m)
                fire_s(b)

                @pl.when(m < chunks - nbuf)
                def _():
                    fire_idx(b, m + nbuf)
                bp = (b + 2) % nbuf

                @pl.when(m >= 2)
                def _():
                    drain_s(bp)

                @pl.when(m < chunks - 2)
                def _():
                    fire_g(bp, m + 2)
            return 0

        lax.fori_loop(0, chunks // nbuf, loop_body, 0)
        drain_s(2)
        drain_s(3)
        plsc.subcore_barrier()

        def co_body(t, _):
            off = s * nrt + t * ce
            pltpu.sync_copy(acc_sh.at[pl.ds(off, ce), :], h0)
            pltpu.sync_copy(den0_sh.at[pl.ds(off, ce)], esa0)
            pltpu.sync_copy(den1_sh.at[pl.ds(off, ce)], eda0)

            def dg_body(g, _):
                dr0 = 1.0 / (esa0[pl.ds(g * 16, 16)] + 1e-16)
                dr1 = 1.0 / (eda0[pl.ds(g * 16, 16)] + 1e-16)
                for i in range(16):
                    r = g * 16 + i
                    h0[r, pl.ds(0, 16)] = lax.broadcast(dr0[i], (16,)) * h0[r, pl.ds(0, 16)]
                    h0[r, pl.ds(16, 16)] = lax.broadcast(dr1[i], (16,)) * h0[r, pl.ds(16, 16)]
                return 0
            lax.fori_loop(0, ce // 16, dg_body, 0)

            @pl.when(c == 0)
            def _():
                pltpu.sync_copy(h0, accA.at[pl.ds(off, ce), :])

            @pl.when(c == 1)
            def _():
                pltpu.sync_copy(h0, accB.at[pl.ds(off, ce), :])
            return 0

        lax.fori_loop(0, nrt // ce, co_body, 0)

    f32 = jnp.float32
    out_type = [jax.ShapeDtypeStruct((npad_acc, 32), f32),
                jax.ShapeDtypeStruct((npad_acc, 32), f32)]
    per_set = [
        pltpu.VMEM((2, 128), jnp.int32),
        pltpu.VMEM((1, 128), jnp.int32),
        pltpu.VMEM((ce,), f32),
        pltpu.VMEM((ce,), f32),
        pltpu.VMEM((ce, 32), f32),
        pltpu.VMEM((ce,), f32),
        pltpu.VMEM((ce,), f32),
    ]
    scratch_types = per_set * 4 + [
        pltpu.VMEM_SHARED((npad_acc, 32), f32),
        pltpu.VMEM_SHARED((npad_den,), f32),
        pltpu.VMEM_SHARED((npad_den,), f32),
    ] + [pltpu.SemaphoreType.DMA] * 12
    mesh = plsc.VectorSubcoreMesh(core_axis_name="c", subcore_axis_name="s",
                                  num_cores=2, num_subcores=ns)
    return pl.kernel(body, out_type=out_type, mesh=mesh,
                     scratch_types=scratch_types,
                     compiler_params=pltpu.CompilerParams(use_tc_tiling_on_sc=False))


_EDGE_ROWS_PAD = 6272
_CHUNKS = 392
_NPAD_ACC = 51200
_NPAD_DEN = 51200


def _edges(hA, hB, es, ed, srdr):
    fn = _build_edge_kernel(NPAD, E, _EDGE_ROWS_PAD, _CHUNKS, _NPAD_ACC, _NPAD_DEN)
    return fn(srdr, es[0], es[1], es[2], es[3], ed[0], ed[1], ed[2], ed[3],
              hA, hB)


def kernel(x, edge_index, fp_W, fp_b, fp_g, fp_beta, ip_W, ip_b,
           gat_W, gat_asrc, gat_adst, gat_bias, proj_W, proj_b, ln_g, ln_b):
    pad = _EDGE_ROWS_PAD * 128 - E
    zpad = jnp.zeros((pad,), jnp.int32)
    srcr = jnp.concatenate([edge_index[0], zpad]).reshape(_EDGE_ROWS_PAD, 1, 128)
    dstr = jnp.concatenate([edge_index[1], zpad]).reshape(_EDGE_ROWS_PAD, 1, 128)
    srdr = jnp.concatenate([srcr, dstr], axis=1)
    Wc = [gat_W[l].transpose(1, 0, 2).reshape(HID, HEADS * HD) for l in range(LAYERS)]
    eye = jnp.repeat(jnp.eye(HEADS, dtype=jnp.float32), HD, axis=0)
    asc = [eye * gat_asrc[l].reshape(HEADS * HD)[:, None] for l in range(LAYERS)]
    adc = [eye * gat_adst[l].reshape(HEADS * HD)[:, None] for l in range(LAYERS)]
    bc = [gat_bias[l].reshape(HEADS * HD) for l in range(LAYERS)]

    x_pad = jnp.zeros((NPAD, D_IN), jnp.float32).at[:N].set(x)
    res = _dense_in(x_pad, fp_W, fp_b, fp_g, fp_beta, ip_W, ip_b,
                    Wc[0], asc[0], adc[0])
    h, hA, hB = res[0], res[1], res[2]
    es, ed = res[3:7], res[7:11]
    for l in range(LAYERS):
        accA, accB = _edges(hA, hB, es, ed, srdr)
        last = l == LAYERS - 1
        nxt = l + 1 if not last else l
        res = _dense_layer(h, accA, accB, bc[l], proj_W[l], proj_b[l],
                           ln_g[l], ln_b[l], Wc[nxt], asc[nxt], adc[nxt], last)
        h = res[0]
        if not last:
            hA, hB = res[1], res[2]
            es, ed = res[3:7], res[7:11]
    return h[:N]
